# baseline (device time: 27507 ns/iter reference)
import functools

import jax
import jax.numpy as jnp
from jax import lax
from jax.experimental import pallas as pl
from jax.experimental.pallas import tpu as pltpu

N_DEV = 8
B, SQ, SKV, DH = 2, 128, 128, 64
H_LOC = 4
D_MODEL = 512
BLK = 64
XOR_PARTNERS = (1, 3, 4)


def kernel(x, Wq, K_ext, V_ext, Wo):
    d_head_loc = Wq.shape[1]

    def body(x_ref, wq_ref, k_hbm, v_hbm, wo_ref, out_ref,
             kloc, vloc, copy_sems, sbuf, comm_ref, send_sems, recv_sems):
        my = lax.axis_index("i")

        h0 = my * H_LOC
        cp_k = pltpu.make_async_copy(
            k_hbm.at[:, :, pl.ds(h0, H_LOC), :], kloc, copy_sems.at[0])
        cp_v = pltpu.make_async_copy(
            v_hbm.at[:, :, pl.ds(h0, H_LOC), :], vloc, copy_sems.at[1])
        cp_k.start()
        cp_v.start()

        barrier = pltpu.get_barrier_semaphore()
        for xr in XOR_PARTNERS:
            pl.semaphore_signal(barrier, inc=1, device_id=(my ^ xr,),
                                device_id_type=pl.DeviceIdType.MESH)
        pl.semaphore_wait(barrier, len(XOR_PARTNERS))

        cp_k.wait()
        cp_v.wait()

        qb = lax.broadcasted_iota(jnp.int32, (SQ, SKV), 0) // BLK
        kb = lax.broadcasted_iota(jnp.int32, (SQ, SKV), 1) // BLK
        mask = kb <= qb

        wq_bf = wq_ref[:].astype(jnp.bfloat16)
        wo_bf = wo_ref[:].astype(jnp.bfloat16)

        def compute_partial(b):
            xb = x_ref[b].astype(jnp.bfloat16)
            q = jnp.dot(xb, wq_bf,
                        preferred_element_type=jnp.float32
                        ).astype(jnp.bfloat16)
            kk = kloc[b].astype(jnp.bfloat16)
            vv = vloc[b].astype(jnp.bfloat16)
            ctxs = []
            for h in range(H_LOC):
                qh = q[:, h * DH:(h + 1) * DH]
                kh = kk[:, h, :]
                vh = vv[:, h, :]
                s = lax.dot_general(
                    qh, kh, (((1,), (1,)), ((), ())),
                    preferred_element_type=jnp.float32) * 0.125
                s = jnp.where(mask, s, -1e9)
                m = jnp.max(s, axis=1, keepdims=True)
                w = jnp.exp(s - m)
                w = w / jnp.sum(w, axis=1, keepdims=True)
                ctxs.append(jnp.dot(w.astype(jnp.bfloat16), vh,
                                    preferred_element_type=jnp.float32
                                    ).astype(jnp.bfloat16))
            ctx = jnp.concatenate(ctxs, axis=1)
            partial = jnp.dot(ctx, wo_bf,
                              preferred_element_type=jnp.float32)
            out_ref[b] = partial
            sbuf[b] = partial.astype(jnp.bfloat16)

        def exchange(r, c):
            return pltpu.make_async_remote_copy(
                src_ref=sbuf.at[c],
                dst_ref=comm_ref.at[r, c],
                send_sem=send_sems.at[r, c],
                recv_sem=recv_sems.at[r, c],
                device_id=(my ^ XOR_PARTNERS[r],),
                device_id_type=pl.DeviceIdType.MESH,
            )

        n_rounds = len(XOR_PARTNERS)
        rd = {}
        compute_partial(0)
        if n_rounds:
            rd[(0, 0)] = exchange(0, 0)
            rd[(0, 0)].start()
        compute_partial(1)
        if n_rounds:
            rd[(0, 1)] = exchange(0, 1)
            rd[(0, 1)].start()
        for r in range(n_rounds):
            for c in range(B):
                rd[(r, c)].wait()
                acc = out_ref[c] + comm_ref[r, c].astype(jnp.float32)
                out_ref[c] = acc
                if r + 1 < n_rounds:
                    sbuf[c] = acc.astype(jnp.bfloat16)
                    rd[(r + 1, c)] = exchange(r + 1, c)
                    rd[(r + 1, c)].start()

        @functools.partial(pl.run_scoped, sem2=pltpu.SemaphoreType.REGULAR)
        def _(sem2):
            for xr in XOR_PARTNERS:
                pl.semaphore_signal(sem2, inc=1, device_id=(my ^ xr,),
                                    device_id_type=pl.DeviceIdType.MESH)
            pl.semaphore_wait(sem2, len(XOR_PARTNERS))

    return pl.pallas_call(
        body,
        out_shape=jax.ShapeDtypeStruct((B, SQ, D_MODEL), jnp.float32),
        in_specs=[
            pl.BlockSpec(memory_space=pltpu.MemorySpace.VMEM),
            pl.BlockSpec(memory_space=pltpu.MemorySpace.VMEM),
            pl.BlockSpec(memory_space=pltpu.MemorySpace.HBM),
            pl.BlockSpec(memory_space=pltpu.MemorySpace.HBM),
            pl.BlockSpec(memory_space=pltpu.MemorySpace.VMEM),
        ],
        out_specs=pl.BlockSpec(memory_space=pltpu.MemorySpace.VMEM),
        scratch_shapes=[
            pltpu.VMEM((B, SQ, H_LOC, DH), jnp.float32),
            pltpu.VMEM((B, SQ, H_LOC, DH), jnp.float32),
            pltpu.SemaphoreType.DMA((2,)),
            pltpu.VMEM((B, SQ, D_MODEL), jnp.bfloat16),
            pltpu.VMEM((3, B, SQ, D_MODEL), jnp.bfloat16),
            pltpu.SemaphoreType.DMA((3, B)),
            pltpu.SemaphoreType.DMA((3, B)),
        ],
        compiler_params=pltpu.CompilerParams(collective_id=0),
    )(x, Wq, K_ext, V_ext, Wo)


# device time: 24357 ns/iter; 1.1293x vs baseline; 1.1293x over previous
import functools

import jax
import jax.numpy as jnp
from jax import lax
from jax.experimental import pallas as pl
from jax.experimental.pallas import tpu as pltpu

N_DEV = 8
B, SQ, SKV, DH = 2, 128, 128, 64
H_LOC = 4
D_MODEL = 512
BLK = 64
XOR_PARTNERS = (1, 3, 4)


def kernel(x, Wq, K_ext, V_ext, Wo):
    d_head_loc = Wq.shape[1]

    K_t = jnp.transpose(K_ext, (0, 2, 3, 1))
    V_t = jnp.transpose(V_ext, (0, 2, 3, 1))

    def body(x_ref, wq_ref, k_hbm, v_hbm, wo_ref, out_ref,
             kloc, vloc, copy_sems, sbuf, comm_ref, send_sems, recv_sems):
        my = lax.axis_index("i")

        h0 = my * H_LOC
        cp_k = pltpu.make_async_copy(
            k_hbm.at[:, pl.ds(h0, H_LOC), :, :], kloc, copy_sems.at[0])
        cp_v = pltpu.make_async_copy(
            v_hbm.at[:, pl.ds(h0, H_LOC), :, :], vloc, copy_sems.at[1])
        cp_k.start()
        cp_v.start()

        barrier = pltpu.get_barrier_semaphore()
        for xr in XOR_PARTNERS:
            pl.semaphore_signal(barrier, inc=1, device_id=(my ^ xr,),
                                device_id_type=pl.DeviceIdType.MESH)
        pl.semaphore_wait(barrier, len(XOR_PARTNERS))

        cp_k.wait()
        cp_v.wait()

        qb = lax.broadcasted_iota(jnp.int32, (SQ, SKV), 0) // BLK
        kb = lax.broadcasted_iota(jnp.int32, (SQ, SKV), 1) // BLK
        mask = kb <= qb

        wq_bf = wq_ref[:].astype(jnp.bfloat16)
        wo_bf = wo_ref[:].astype(jnp.bfloat16)

        def compute_partial(b):
            xb = x_ref[b].astype(jnp.bfloat16)
            q = jnp.dot(xb, wq_bf,
                        preferred_element_type=jnp.float32
                        ).astype(jnp.bfloat16)
            kk = kloc[b].astype(jnp.bfloat16)
            vv = vloc[b].astype(jnp.bfloat16)
            ctxs = []
            for h in range(H_LOC):
                qh = q[:, h * DH:(h + 1) * DH]
                kh = kk[h]
                vh = vv[h]
                s = jnp.dot(qh, kh,
                            preferred_element_type=jnp.float32) * 0.125
                s = jnp.where(mask, s, -1e9)
                m = jnp.max(s, axis=1, keepdims=True)
                w = jnp.exp(s - m)
                w = w / jnp.sum(w, axis=1, keepdims=True)
                ctxs.append(lax.dot_general(
                    w.astype(jnp.bfloat16), vh, (((1,), (1,)), ((), ())),
                    preferred_element_type=jnp.float32
                    ).astype(jnp.bfloat16))
            ctx = jnp.concatenate(ctxs, axis=1)
            partial = jnp.dot(ctx, wo_bf,
                              preferred_element_type=jnp.float32)
            out_ref[b] = partial
            sbuf[b] = partial.astype(jnp.bfloat16)

        def exchange(r, c):
            return pltpu.make_async_remote_copy(
                src_ref=sbuf.at[c],
                dst_ref=comm_ref.at[r, c],
                send_sem=send_sems.at[r, c],
                recv_sem=recv_sems.at[r, c],
                device_id=(my ^ XOR_PARTNERS[r],),
                device_id_type=pl.DeviceIdType.MESH,
            )

        n_rounds = len(XOR_PARTNERS)
        rd = {}
        compute_partial(0)
        if n_rounds:
            rd[(0, 0)] = exchange(0, 0)
            rd[(0, 0)].start()
        compute_partial(1)
        if n_rounds:
            rd[(0, 1)] = exchange(0, 1)
            rd[(0, 1)].start()
        for r in range(n_rounds):
            for c in range(B):
                rd[(r, c)].wait()
                acc = out_ref[c] + comm_ref[r, c].astype(jnp.float32)
                out_ref[c] = acc
                if r + 1 < n_rounds:
                    sbuf[c] = acc.astype(jnp.bfloat16)
                    rd[(r + 1, c)] = exchange(r + 1, c)
                    rd[(r + 1, c)].start()

        @functools.partial(pl.run_scoped, sem2=pltpu.SemaphoreType.REGULAR)
        def _(sem2):
            for xr in XOR_PARTNERS:
                pl.semaphore_signal(sem2, inc=1, device_id=(my ^ xr,),
                                    device_id_type=pl.DeviceIdType.MESH)
            pl.semaphore_wait(sem2, len(XOR_PARTNERS))

    return pl.pallas_call(
        body,
        out_shape=jax.ShapeDtypeStruct((B, SQ, D_MODEL), jnp.float32),
        in_specs=[
            pl.BlockSpec(memory_space=pltpu.MemorySpace.VMEM),
            pl.BlockSpec(memory_space=pltpu.MemorySpace.VMEM),
            pl.BlockSpec(memory_space=pltpu.MemorySpace.HBM),
            pl.BlockSpec(memory_space=pltpu.MemorySpace.HBM),
            pl.BlockSpec(memory_space=pltpu.MemorySpace.VMEM),
        ],
        out_specs=pl.BlockSpec(memory_space=pltpu.MemorySpace.VMEM),
        scratch_shapes=[
            pltpu.VMEM((B, H_LOC, DH, SKV), jnp.float32),
            pltpu.VMEM((B, H_LOC, DH, SKV), jnp.float32),
            pltpu.SemaphoreType.DMA((2,)),
            pltpu.VMEM((B, SQ, D_MODEL), jnp.bfloat16),
            pltpu.VMEM((3, B, SQ, D_MODEL), jnp.bfloat16),
            pltpu.SemaphoreType.DMA((3, B)),
            pltpu.SemaphoreType.DMA((3, B)),
        ],
        compiler_params=pltpu.CompilerParams(collective_id=0),
    )(x, Wq, K_t, V_t, Wo)
